# baseline (speedup 1.0000x reference)
import functools

import jax
import jax.numpy as jnp
from jax.experimental import pallas as pl
from jax.experimental.pallas import tpu as pltpu

_LEAKY = 0.5


def _pass1_body(adj_ref, iemb_ref, uembT_ref, ti_ref, tuT_ref, hiT_ref, adj8_ref,
                *, nsteps):
    r = pl.program_id(0)

    @pl.when(r == 0)
    def _init():
        tuT_ref[...] = jnp.zeros_like(tuT_ref)
        hiT_ref[...] = jnp.zeros_like(hiT_ref)

    adj = adj_ref[...]
    adjb = adj.astype(jnp.bfloat16)
    adj8_ref[...] = adj.astype(jnp.int8)

    ti = jnp.dot(adjb, iemb_ref[...].astype(jnp.bfloat16),
                 preferred_element_type=jnp.float32)
    ti_ref[...] = ti
    tuT_ref[...] += jnp.dot(uembT_ref[...].astype(jnp.bfloat16), adjb,
                            preferred_element_type=jnp.float32)
    hiT_ref[...] += jnp.dot(ti.astype(jnp.bfloat16).T, adjb,
                            preferred_element_type=jnp.float32)

    @pl.when(r == nsteps - 1)
    def _act():
        hi = hiT_ref[...]
        hiT_ref[...] = jnp.where(hi >= 0, hi, _LEAKY * hi)


def _pass2_body(adj8_ref, tub_ref, hu_ref, *, d):
    p = jnp.dot(adj8_ref[...].astype(jnp.bfloat16), tub_ref[...],
                preferred_element_type=jnp.float32)
    nchunk = p.shape[1] // d
    hu = p[:, :d]
    for c in range(1, nchunk):
        hu = hu + p[:, c * d:(c + 1) * d]
    hu_ref[...] = jnp.where(hu >= 0, hu, _LEAKY * hu)


@functools.partial(jax.jit, static_argnames=("stripe",))
def _dhcf(adj, user_emb, item_emb, stripe=256):
    n_u, n_i = adj.shape
    d = user_emb.shape[1]
    nsteps = n_u // stripe

    grid = (nsteps,)
    params = pltpu.CompilerParams(dimension_semantics=("arbitrary",))

    t_i, t_uT, h_iT, adj8 = pl.pallas_call(
        functools.partial(_pass1_body, nsteps=nsteps),
        grid=grid,
        in_specs=[
            pl.BlockSpec((stripe, n_i), lambda r: (r, 0)),
            pl.BlockSpec((n_i, d), lambda r: (0, 0)),
            pl.BlockSpec((d, stripe), lambda r: (0, r)),
        ],
        out_specs=[
            pl.BlockSpec((stripe, d), lambda r: (r, 0)),
            pl.BlockSpec((d, n_i), lambda r: (0, 0)),
            pl.BlockSpec((d, n_i), lambda r: (0, 0)),
            pl.BlockSpec((stripe, n_i), lambda r: (r, 0)),
        ],
        out_shape=[
            jax.ShapeDtypeStruct((n_u, d), jnp.float32),
            jax.ShapeDtypeStruct((d, n_i), jnp.float32),
            jax.ShapeDtypeStruct((d, n_i), jnp.float32),
            jax.ShapeDtypeStruct((n_u, n_i), jnp.int8),
        ],
        compiler_params=params,
    )(adj, item_emb, user_emb.T)

    nchunk = 128 // d
    chunk = n_i // nchunk
    t_u = t_uT.T.astype(jnp.bfloat16)
    t_u_blockdiag = jnp.concatenate(
        [jnp.pad(t_u[c * chunk:(c + 1) * chunk], ((0, 0), (c * d, 128 - (c + 1) * d)))
         for c in range(nchunk)], axis=0)

    h_u = pl.pallas_call(
        functools.partial(_pass2_body, d=d),
        grid=grid,
        in_specs=[
            pl.BlockSpec((stripe, n_i), lambda r: (r, 0)),
            pl.BlockSpec((n_i, 128), lambda r: (0, 0)),
        ],
        out_specs=pl.BlockSpec((stripe, d), lambda r: (r, 0)),
        out_shape=jax.ShapeDtypeStruct((n_u, d), jnp.float32),
        compiler_params=params,
    )(adj8, t_u_blockdiag)

    h_i = h_iT.T
    user_all = jnp.concatenate([user_emb, h_u, h_u], axis=1)
    item_all = jnp.concatenate([item_emb, h_i, h_i], axis=1)
    return user_all, item_all


def kernel(adj, user_emb, item_emb):
    return _dhcf(adj, user_emb, item_emb)


# speedup vs baseline: 1.0043x; 1.0043x over previous
import functools

import jax
import jax.numpy as jnp
from jax.experimental import pallas as pl
from jax.experimental.pallas import tpu as pltpu

_LEAKY = 0.5


def _pass1_body(adj_ref, iemb_ref, uembT_ref, tuT_ref, hiT_ref, adj8_ref,
                *, nsteps):
    r = pl.program_id(0)

    @pl.when(r == 0)
    def _init():
        tuT_ref[...] = jnp.zeros_like(tuT_ref)
        hiT_ref[...] = jnp.zeros_like(hiT_ref)

    adj = adj_ref[...]
    adjb = adj.astype(jnp.bfloat16)
    adj8_ref[...] = adj.astype(jnp.int8)

    ti = jnp.dot(adjb, iemb_ref[...].astype(jnp.bfloat16),
                 preferred_element_type=jnp.float32)
    tuT_ref[...] += jnp.dot(uembT_ref[...].astype(jnp.bfloat16), adjb,
                            preferred_element_type=jnp.float32)
    hiT_ref[...] += jnp.dot(ti.astype(jnp.bfloat16).T, adjb,
                            preferred_element_type=jnp.float32)

    @pl.when(r == nsteps - 1)
    def _act():
        hi = hiT_ref[...]
        hiT_ref[...] = jnp.where(hi >= 0, hi, _LEAKY * hi)


def _pass2_body(adj8_ref, tu_ref, uemb_ref, iemb_ref, hiT_ref,
                uall_ref, iall_ref, *, d):
    hu = jnp.dot(adj8_ref[...].astype(jnp.bfloat16),
                 tu_ref[...].astype(jnp.bfloat16),
                 preferred_element_type=jnp.float32)
    hu = jnp.where(hu >= 0, hu, _LEAKY * hu)
    uall_ref[:, :d] = uemb_ref[...]
    uall_ref[:, d:2 * d] = hu
    uall_ref[:, 2 * d:] = hu
    hi = hiT_ref[...].T
    iall_ref[:, :d] = iemb_ref[...]
    iall_ref[:, d:2 * d] = hi
    iall_ref[:, 2 * d:] = hi


@functools.partial(jax.jit, static_argnames=("stripe",))
def _dhcf(adj, user_emb, item_emb, stripe=256):
    n_u, n_i = adj.shape
    d = user_emb.shape[1]
    nsteps = n_u // stripe

    grid = (nsteps,)
    params = pltpu.CompilerParams(dimension_semantics=("arbitrary",))

    t_uT, h_iT, adj8 = pl.pallas_call(
        functools.partial(_pass1_body, nsteps=nsteps),
        grid=grid,
        in_specs=[
            pl.BlockSpec((stripe, n_i), lambda r: (r, 0)),
            pl.BlockSpec((n_i, d), lambda r: (0, 0)),
            pl.BlockSpec((d, stripe), lambda r: (0, r)),
        ],
        out_specs=[
            pl.BlockSpec((d, n_i), lambda r: (0, 0)),
            pl.BlockSpec((d, n_i), lambda r: (0, 0)),
            pl.BlockSpec((stripe, n_i), lambda r: (r, 0)),
        ],
        out_shape=[
            jax.ShapeDtypeStruct((d, n_i), jnp.float32),
            jax.ShapeDtypeStruct((d, n_i), jnp.float32),
            jax.ShapeDtypeStruct((n_u, n_i), jnp.int8),
        ],
        compiler_params=params,
    )(adj, item_emb, user_emb.T)

    user_all, item_all = pl.pallas_call(
        functools.partial(_pass2_body, d=d),
        grid=grid,
        in_specs=[
            pl.BlockSpec((stripe, n_i), lambda r: (r, 0)),
            pl.BlockSpec((n_i, d), lambda r: (0, 0)),
            pl.BlockSpec((stripe, d), lambda r: (r, 0)),
            pl.BlockSpec((stripe, d), lambda r: (r, 0)),
            pl.BlockSpec((d, stripe), lambda r: (0, r)),
        ],
        out_specs=[
            pl.BlockSpec((stripe, 3 * d), lambda r: (r, 0)),
            pl.BlockSpec((stripe, 3 * d), lambda r: (r, 0)),
        ],
        out_shape=[
            jax.ShapeDtypeStruct((n_u, 3 * d), jnp.float32),
            jax.ShapeDtypeStruct((n_i, 3 * d), jnp.float32),
        ],
        compiler_params=params,
    )(adj8, t_uT.T, user_emb, item_emb, h_iT)

    return user_all, item_all


def kernel(adj, user_emb, item_emb):
    return _dhcf(adj, user_emb, item_emb)


# speedup vs baseline: 1.0411x; 1.0366x over previous
import functools

import jax
import jax.numpy as jnp
from jax.experimental import pallas as pl
from jax.experimental.pallas import tpu as pltpu

_LEAKY = 0.5


def _pass1_body(adj_ref, iemb_ref, uembT_ref, tuT_ref, hiT_ref, adj8_ref,
                *, nsteps):
    r = pl.program_id(0)

    @pl.when(r == 0)
    def _init():
        tuT_ref[...] = jnp.zeros_like(tuT_ref)
        hiT_ref[...] = jnp.zeros_like(hiT_ref)

    adj = adj_ref[...]
    adjb = adj.astype(jnp.bfloat16)
    adj8_ref[...] = adj.astype(jnp.int8)

    ti = jnp.dot(adjb, iemb_ref[...].astype(jnp.bfloat16),
                 preferred_element_type=jnp.float32)
    tuT_ref[...] += jnp.dot(uembT_ref[...].astype(jnp.bfloat16), adjb,
                            preferred_element_type=jnp.float32)
    hiT_ref[...] += jnp.dot(ti.astype(jnp.bfloat16).T, adjb,
                            preferred_element_type=jnp.float32)

    @pl.when(r == nsteps - 1)
    def _act():
        hi = hiT_ref[...]
        hiT_ref[...] = jnp.where(hi >= 0, hi, _LEAKY * hi)


def _pass2_body(adj8_ref, tu_ref, hu_ref):
    hu = jnp.dot(adj8_ref[...].astype(jnp.bfloat16),
                 tu_ref[...].astype(jnp.bfloat16),
                 preferred_element_type=jnp.float32)
    hu_ref[...] = jnp.where(hu >= 0, hu, _LEAKY * hu)


@functools.partial(jax.jit, static_argnames=("stripe", "stripe2"))
def _dhcf(adj, user_emb, item_emb, stripe=256, stripe2=1024):
    n_u, n_i = adj.shape
    d = user_emb.shape[1]
    nsteps = n_u // stripe

    params = pltpu.CompilerParams(dimension_semantics=("arbitrary",))

    t_uT, h_iT, adj8 = pl.pallas_call(
        functools.partial(_pass1_body, nsteps=nsteps),
        grid=(nsteps,),
        in_specs=[
            pl.BlockSpec((stripe, n_i), lambda r: (r, 0)),
            pl.BlockSpec((n_i, d), lambda r: (0, 0)),
            pl.BlockSpec((d, stripe), lambda r: (0, r)),
        ],
        out_specs=[
            pl.BlockSpec((d, n_i), lambda r: (0, 0)),
            pl.BlockSpec((d, n_i), lambda r: (0, 0)),
            pl.BlockSpec((stripe, n_i), lambda r: (r, 0)),
        ],
        out_shape=[
            jax.ShapeDtypeStruct((d, n_i), jnp.float32),
            jax.ShapeDtypeStruct((d, n_i), jnp.float32),
            jax.ShapeDtypeStruct((n_u, n_i), jnp.int8),
        ],
        compiler_params=params,
    )(adj, item_emb, user_emb.T)

    h_u = pl.pallas_call(
        _pass2_body,
        grid=(n_u // stripe2,),
        in_specs=[
            pl.BlockSpec((stripe2, n_i), lambda r: (r, 0)),
            pl.BlockSpec((n_i, d), lambda r: (0, 0)),
        ],
        out_specs=pl.BlockSpec((stripe2, d), lambda r: (r, 0)),
        out_shape=jax.ShapeDtypeStruct((n_u, d), jnp.float32),
        compiler_params=params,
    )(adj8, t_uT.T)

    h_i = h_iT.T
    user_all = jnp.concatenate([user_emb, h_u, h_u], axis=1)
    item_all = jnp.concatenate([item_emb, h_i, h_i], axis=1)
    return user_all, item_all


def kernel(adj, user_emb, item_emb):
    return _dhcf(adj, user_emb, item_emb)


# speedup vs baseline: 1.0457x; 1.0044x over previous
import functools

import jax
import jax.numpy as jnp
from jax.experimental import pallas as pl
from jax.experimental.pallas import tpu as pltpu

_LEAKY = 0.5


def _pass1_body(adj_ref, iemb_ref, uemb_ref, tuT_ref, hiT_ref, adj8_ref,
                *, nsteps):
    r = pl.program_id(0)

    @pl.when(r == 0)
    def _init():
        tuT_ref[...] = jnp.zeros_like(tuT_ref)
        hiT_ref[...] = jnp.zeros_like(hiT_ref)

    adj = adj_ref[...]
    adjb = adj.astype(jnp.bfloat16)
    adj8_ref[...] = adj.astype(jnp.int8)

    ti = jnp.dot(adjb, iemb_ref[...].astype(jnp.bfloat16),
                 preferred_element_type=jnp.float32)
    tuT_ref[...] += jax.lax.dot_general(
        uemb_ref[...].astype(jnp.bfloat16), adjb, (((0,), (0,)), ((), ())),
        preferred_element_type=jnp.float32)
    hiT_ref[...] += jax.lax.dot_general(
        ti.astype(jnp.bfloat16), adjb, (((0,), (0,)), ((), ())),
        preferred_element_type=jnp.float32)

    @pl.when(r == nsteps - 1)
    def _act():
        hi = hiT_ref[...]
        hiT_ref[...] = jnp.where(hi >= 0, hi, _LEAKY * hi)


def _pass2_body(adj8_ref, tuT_ref, hu_ref):
    hu = jax.lax.dot_general(
        adj8_ref[...].astype(jnp.bfloat16), tuT_ref[...].astype(jnp.bfloat16),
        (((1,), (1,)), ((), ())), preferred_element_type=jnp.float32)
    hu_ref[...] = jnp.where(hu >= 0, hu, _LEAKY * hu)


@functools.partial(jax.jit, static_argnames=("stripe", "stripe2"))
def _dhcf(adj, user_emb, item_emb, stripe=256, stripe2=1024):
    n_u, n_i = adj.shape
    d = user_emb.shape[1]
    nsteps = n_u // stripe

    params = pltpu.CompilerParams(dimension_semantics=("arbitrary",))

    t_uT, h_iT, adj8 = pl.pallas_call(
        functools.partial(_pass1_body, nsteps=nsteps),
        grid=(nsteps,),
        in_specs=[
            pl.BlockSpec((stripe, n_i), lambda r: (r, 0)),
            pl.BlockSpec((n_i, d), lambda r: (0, 0)),
            pl.BlockSpec((stripe, d), lambda r: (r, 0)),
        ],
        out_specs=[
            pl.BlockSpec((d, n_i), lambda r: (0, 0)),
            pl.BlockSpec((d, n_i), lambda r: (0, 0)),
            pl.BlockSpec((stripe, n_i), lambda r: (r, 0)),
        ],
        out_shape=[
            jax.ShapeDtypeStruct((d, n_i), jnp.float32),
            jax.ShapeDtypeStruct((d, n_i), jnp.float32),
            jax.ShapeDtypeStruct((n_u, n_i), jnp.int8),
        ],
        compiler_params=params,
    )(adj, item_emb, user_emb)

    h_u = pl.pallas_call(
        _pass2_body,
        grid=(n_u // stripe2,),
        in_specs=[
            pl.BlockSpec((stripe2, n_i), lambda r: (r, 0)),
            pl.BlockSpec((d, n_i), lambda r: (0, 0)),
        ],
        out_specs=pl.BlockSpec((stripe2, d), lambda r: (r, 0)),
        out_shape=jax.ShapeDtypeStruct((n_u, d), jnp.float32),
        compiler_params=params,
    )(adj8, t_uT)

    h_i = h_iT.T
    user_all = jnp.concatenate([user_emb, h_u, h_u], axis=1)
    item_all = jnp.concatenate([item_emb, h_i, h_i], axis=1)
    return user_all, item_all


def kernel(adj, user_emb, item_emb):
    return _dhcf(adj, user_emb, item_emb)


# speedup vs baseline: 1.0591x; 1.0129x over previous
import functools

import jax
import jax.numpy as jnp
from jax.experimental import pallas as pl
from jax.experimental.pallas import tpu as pltpu

_LEAKY = 0.5


def _pass1_body(adj_ref, iemb_ref, uemb_ref, acc_ref, adj8_ref, *, nsteps, d):
    r = pl.program_id(0)

    @pl.when(r == 0)
    def _init():
        acc_ref[...] = jnp.zeros_like(acc_ref)

    adj = adj_ref[...]
    adjb = adj.astype(jnp.bfloat16)
    adj8_ref[...] = adj.astype(jnp.int8)

    ti = jnp.dot(adjb, iemb_ref[...].astype(jnp.bfloat16),
                 preferred_element_type=jnp.float32)
    x = jnp.concatenate([uemb_ref[...], ti], axis=1).astype(jnp.bfloat16)
    acc_ref[...] += jax.lax.dot_general(
        x, adjb, (((0,), (0,)), ((), ())), preferred_element_type=jnp.float32)

    @pl.when(r == nsteps - 1)
    def _act():
        hi = acc_ref[d:, :]
        acc_ref[d:, :] = jnp.where(hi >= 0, hi, _LEAKY * hi)


def _pass2_body(adj8_ref, tuT_ref, hu_ref):
    hu = jax.lax.dot_general(
        adj8_ref[...].astype(jnp.bfloat16), tuT_ref[...].astype(jnp.bfloat16),
        (((1,), (1,)), ((), ())), preferred_element_type=jnp.float32)
    hu_ref[...] = jnp.where(hu >= 0, hu, _LEAKY * hu)


@functools.partial(jax.jit, static_argnames=("stripe", "stripe2"))
def _dhcf(adj, user_emb, item_emb, stripe=256, stripe2=1024):
    n_u, n_i = adj.shape
    d = user_emb.shape[1]
    nsteps = n_u // stripe

    params = pltpu.CompilerParams(dimension_semantics=("arbitrary",))

    acc, adj8 = pl.pallas_call(
        functools.partial(_pass1_body, nsteps=nsteps, d=d),
        grid=(nsteps,),
        in_specs=[
            pl.BlockSpec((stripe, n_i), lambda r: (r, 0)),
            pl.BlockSpec((n_i, d), lambda r: (0, 0)),
            pl.BlockSpec((stripe, d), lambda r: (r, 0)),
        ],
        out_specs=[
            pl.BlockSpec((2 * d, n_i), lambda r: (0, 0)),
            pl.BlockSpec((stripe, n_i), lambda r: (r, 0)),
        ],
        out_shape=[
            jax.ShapeDtypeStruct((2 * d, n_i), jnp.float32),
            jax.ShapeDtypeStruct((n_u, n_i), jnp.int8),
        ],
        compiler_params=params,
    )(adj, item_emb, user_emb)

    t_uT = acc[:d]
    h_i = acc[d:].T

    h_u = pl.pallas_call(
        _pass2_body,
        grid=(n_u // stripe2,),
        in_specs=[
            pl.BlockSpec((stripe2, n_i), lambda r: (r, 0)),
            pl.BlockSpec((d, n_i), lambda r: (0, 0)),
        ],
        out_specs=pl.BlockSpec((stripe2, d), lambda r: (r, 0)),
        out_shape=jax.ShapeDtypeStruct((n_u, d), jnp.float32),
        compiler_params=params,
    )(adj8, t_uT)

    user_all = jnp.concatenate([user_emb, h_u, h_u], axis=1)
    item_all = jnp.concatenate([item_emb, h_i, h_i], axis=1)
    return user_all, item_all


def kernel(adj, user_emb, item_emb):
    return _dhcf(adj, user_emb, item_emb)
